# baseline (speedup 1.0000x reference)
import functools

import jax
import jax.numpy as jnp
from jax import lax
from jax.experimental import pallas as pl
from jax.experimental.pallas import tpu as pltpu
from jax.experimental.pallas import tpu_sc as plsc

D = 128
N = 512
TOTAL = N * N
NW = 32
PER_W = TOTAL // NW
CHUNK = 64
NCHUNK = PER_W // CHUNK
NBUF = 8
L = 16


def _make_kernel():
    mesh = plsc.VectorSubcoreMesh(core_axis_name="c", subcore_axis_name="s")

    scratch = [
        pltpu.VMEM((PER_W,), jnp.int32),
        pltpu.VMEM((PER_W,), jnp.int32),
        pltpu.VMEM_SHARED((65, D), jnp.float32),
    ]
    scratch += [pltpu.VMEM((NBUF * CHUNK, D), jnp.float32)]
    scratch += [pltpu.SemaphoreType.DMA for _ in range(NBUF + NBUF // 2)]

    @functools.partial(
        pl.kernel,
        mesh=mesh,
        out_type=jax.ShapeDtypeStruct((TOTAL, D), jnp.float32),
        scratch_types=scratch,
    )
    def k(templ_hbm, mask_hbm, table_hbm, out_hbm, templ_v, idx_v, table_v,
          *bufs_and_sems):
        rows_all = bufs_and_sems[0]
        gsem = bufs_and_sems[1:1 + NBUF]
        ssem = bufs_and_sems[1 + NBUF:]
        wid = lax.axis_index("s") * 2 + lax.axis_index("c")
        base = wid * PER_W

        @pl.when(lax.axis_index("s") == 0)
        def _():
            pltpu.sync_copy(table_hbm, table_v)

        th = pltpu.async_copy(
            templ_hbm.at[pl.ds(base, PER_W)], templ_v, gsem[0])
        mh = pltpu.async_copy(
            mask_hbm.at[pl.ds(base, PER_W)], idx_v, gsem[1])
        th.wait()
        mh.wait()
        plsc.subcore_barrier()

        def compute_idx_span(e0):
            def body(j, carry):
                o = e0 + j * L
                t = templ_v[pl.ds(o, L)]
                m = idx_v[pl.ds(o, L)]
                idx_v[pl.ds(o, L)] = m * (t + 1)
                return carry
            lax.fori_loop(0, NBUF * CHUNK // L, body, 0)

        compute_idx_span(0)

        @pl.loop(0, NCHUNK, step=NBUF)
        def _(c0):
            handles = []
            for b in range(NBUF):
                if b % 2 == 0:
                    @pl.when(c0 > 0)
                    def _():
                        pltpu.make_async_copy(
                            rows_all.at[pl.ds(0, 2 * CHUNK)],
                            out_hbm.at[pl.ds(0, 2 * CHUNK)], ssem[b // 2]
                        ).wait()
                idx_c = idx_v.at[pl.ds((c0 + b) * CHUNK, CHUNK)]
                handles.append(
                    pltpu.async_copy(
                        table_v.at[idx_c],
                        rows_all.at[pl.ds(b * CHUNK, CHUNK)], gsem[b]))

            @pl.when(c0 + NBUF < NCHUNK)
            def _():
                compute_idx_span((c0 + NBUF) * CHUNK)

            for p in range(NBUF // 2):
                handles[2 * p].wait()
                handles[2 * p + 1].wait()
                pltpu.async_copy(
                    rows_all.at[pl.ds(2 * p * CHUNK, 2 * CHUNK)],
                    out_hbm.at[pl.ds(base + (c0 + 2 * p) * CHUNK, 2 * CHUNK)],
                    ssem[p],
                )
        for p in range(NBUF // 2):
            pltpu.make_async_copy(
                rows_all.at[pl.ds(0, 2 * CHUNK)],
                out_hbm.at[pl.ds(0, 2 * CHUNK)], ssem[p]
            ).wait()

    return k


_embed = _make_kernel()


def kernel(conditional_templ, conditional_templ_mask, table):
    out = _embed(conditional_templ.reshape(TOTAL),
                 conditional_templ_mask.reshape(TOTAL),
                 table)
    return out.reshape(N, N, D)
